# baseline (speedup 1.0000x reference)
import jax
import jax.numpy as jnp
from jax import lax
from jax.experimental import pallas as pl
from jax.experimental.pallas import tpu as pltpu
from jax.experimental.pallas import tpu_sc as plsc

_PLACEHOLDER = 265
_B, _N, _D = 1024, 77, 768
_NW = 32
_BPW = _B // _NW
_PROWS = 56
_NPIECE = _BPW * _N // _PROWS


def _body(tok_hbm, emb_hbm, ph_hbm, out_hbm, tok_v, ph_v, fr_v, buf,
          si0, si1, so0, so1, ssc):
    wid = lax.axis_index("s") * 2 + lax.axis_index("c")
    base = wid * _BPW
    si = (si0, si1)
    so = (so0, so1)
    iota = lax.iota(jnp.int32, 16)

    pltpu.sync_copy(tok_hbm.at[pl.ds(base, _BPW)], tok_v)
    pltpu.sync_copy(ph_hbm.at[pl.ds(base, _BPW)], ph_v)

    for g in range(_BPW // 16):
        colv = jnp.zeros((16,), jnp.int32)
        rows = g * 16 + iota
        for c in range(_N):
            t = plsc.load_gather(tok_v, [rows, jnp.full((16,), c, jnp.int32)])
            colv = colv + jnp.where(t == _PLACEHOLDER, c, 0)
        fr_v[pl.ds(g * 16, 16)] = (base + rows) * _N + colv

    flat_base = base * _N

    def in_cp(p):
        return pltpu.make_async_copy(
            emb_hbm.at[pl.ds(flat_base + p * _PROWS, _PROWS)], buf.at[p % 2], si[p % 2]
        )

    def out_cp(p):
        return pltpu.make_async_copy(
            buf.at[p % 2], out_hbm.at[pl.ds(flat_base + p * _PROWS, _PROWS)], so[p % 2]
        )

    in_cp(0).start()
    in_cp(1).start()
    for p in range(_NPIECE):
        in_cp(p).wait()
        out_cp(p).start()
        if p + 2 < _NPIECE:
            out_cp(p).wait()
            in_cp(p + 2).start()
    out_cp(_NPIECE - 2).wait()
    out_cp(_NPIECE - 1).wait()

    pltpu.make_async_copy(ph_v, out_hbm.at[fr_v], ssc).start()
    pltpu.make_async_copy(ph_v, out_hbm.at[fr_v], ssc).wait()


def kernel(tokenized_text, embedded_text, placeholder_embedding):
    mesh = plsc.VectorSubcoreMesh(core_axis_name="c", subcore_axis_name="s")
    run = pl.kernel(
        _body,
        out_type=jax.ShapeDtypeStruct((_B * _N, _D), embedded_text.dtype),
        mesh=mesh,
        compiler_params=pltpu.CompilerParams(needs_layout_passes=False),
        scratch_types=[
            pltpu.VMEM((_BPW, _N), jnp.int32),
            pltpu.VMEM((_BPW, _D), jnp.float32),
            pltpu.VMEM((_BPW,), jnp.int32),
            pltpu.VMEM((2, _PROWS, _D), jnp.float32),
            pltpu.SemaphoreType.DMA,
            pltpu.SemaphoreType.DMA,
            pltpu.SemaphoreType.DMA,
            pltpu.SemaphoreType.DMA,
            pltpu.SemaphoreType.DMA,
        ],
    )
    out_flat = run(
        tokenized_text, embedded_text.reshape(_B * _N, _D), placeholder_embedding
    )
    return out_flat.reshape(_B, _N, _D)
